# Initial kernel scaffold; baseline (speedup 1.0000x reference)
import jax
import jax.numpy as jnp
from jax.experimental import pallas as pl


def kernel(in_feat, edge_index, node_graph_ids, W, b):
    raise NotImplementedError("write your pallas kernel here")


# speedup vs baseline: 9.7571x; 9.7571x over previous
import functools

import jax
import jax.numpy as jnp
from jax import lax
from jax.experimental import pallas as pl
from jax.experimental.pallas import tpu as pltpu
from jax.experimental.pallas import tpu_sc as plsc

NC = 2
NS = 16
NW = NC * NS
LANES = 16
CHUNK = 128
G = 128


def _sc_mesh():
    return plsc.VectorSubcoreMesh(
        core_axis_name="c", subcore_axis_name="s", num_cores=NC, num_subcores=NS
    )


_SC_PARAMS = pltpu.CompilerParams(use_tc_tiling_on_sc=False)


def _fill(ref, rows, value):
    def body(i, _):
        ref[i] = jnp.full((LANES,), value, jnp.float32)
        return 0

    lax.fori_loop(0, rows, body, 0)


def _make_hist_kernel(nr, j_chunks):
    rt = nr // NS

    @functools.partial(
        pl.kernel,
        out_type=(
            jax.ShapeDtypeStruct((NC, nr, LANES), jnp.float32),
            jax.ShapeDtypeStruct((NC, nr, LANES), jnp.float32),
        ),
        mesh=_sc_mesh(),
        compiler_params=_SC_PARAMS,
        scratch_types=[
            pltpu.VMEM((j_chunks, CHUNK), jnp.int32),
            pltpu.VMEM((j_chunks, CHUNK), jnp.int32),
            pltpu.VMEM((CHUNK, LANES), jnp.float32),
            pltpu.VMEM((rt, LANES), jnp.float32),
            pltpu.VMEM_SHARED((nr, LANES), jnp.float32),
            pltpu.VMEM_SHARED((nr, LANES), jnp.float32),
        ],
    )
    def hist_kernel(src_hbm, dst_hbm, out0_hbm, out1_hbm,
                    src_v, dst_v, ones_v, zero_v, h0_sh, h1_sh):
        c = lax.axis_index("c")
        s = lax.axis_index("s")
        wid = c * NS + s
        base = s * rt

        _fill(ones_v, CHUNK, 1.0)
        _fill(zero_v, rt, 0.0)
        pltpu.sync_copy(zero_v, h0_sh.at[pl.ds(base, rt)])
        pltpu.sync_copy(zero_v, h1_sh.at[pl.ds(base, rt)])
        plsc.subcore_barrier()

        pltpu.sync_copy(src_hbm.at[wid], src_v)
        pltpu.sync_copy(dst_hbm.at[wid], dst_v)

        def chunk(j, _):
            pltpu.sync_copy(ones_v, h0_sh.at[src_v.at[j]], add=True)
            pltpu.sync_copy(ones_v, h1_sh.at[dst_v.at[j]], add=True)
            return 0

        lax.fori_loop(0, j_chunks, chunk, 0)
        plsc.subcore_barrier()

        pltpu.sync_copy(h0_sh.at[pl.ds(base, rt)], out0_hbm.at[c, pl.ds(base, rt)])
        pltpu.sync_copy(h1_sh.at[pl.ds(base, rt)], out1_hbm.at[c, pl.ds(base, rt)])

    return hist_kernel


def _make_scatter_kernel(nr, j_chunks):
    rt = nr // NS

    @functools.partial(
        pl.kernel,
        out_type=jax.ShapeDtypeStruct((NC, nr, LANES), jnp.float32),
        mesh=_sc_mesh(),
        compiler_params=_SC_PARAMS,
        scratch_types=[
            pltpu.VMEM((j_chunks, CHUNK), jnp.int32),
            pltpu.VMEM((j_chunks, CHUNK), jnp.int32),
            pltpu.VMEM((CHUNK, LANES), jnp.float32),
            pltpu.VMEM((rt, LANES), jnp.float32),
            pltpu.VMEM_SHARED((nr, LANES), jnp.float32),
        ],
    )
    def scatter_kernel(h_hbm, src_hbm, dst_hbm, out_hbm,
                       src_v, dst_v, rows_v, zero_v, agg_sh):
        c = lax.axis_index("c")
        s = lax.axis_index("s")
        wid = c * NS + s
        base = s * rt

        _fill(zero_v, rt, 0.0)
        pltpu.sync_copy(zero_v, agg_sh.at[pl.ds(base, rt)])
        plsc.subcore_barrier()

        pltpu.sync_copy(src_hbm.at[wid], src_v)
        pltpu.sync_copy(dst_hbm.at[wid], dst_v)

        def chunk(j, _):
            pltpu.sync_copy(h_hbm.at[src_v.at[j]], rows_v)
            pltpu.sync_copy(rows_v, agg_sh.at[dst_v.at[j]], add=True)
            return 0

        lax.fori_loop(0, j_chunks, chunk, 0)
        plsc.subcore_barrier()

        pltpu.sync_copy(agg_sh.at[pl.ds(base, rt)], out_hbm.at[c, pl.ds(base, rt)])

    return scatter_kernel


def _matmul_body(deg_ref, x_ref, w_ref, o_ref):
    d = deg_ref[0] + deg_ref[1]
    norm = lax.rsqrt(jnp.maximum(d, 1.0))
    xs = x_ref[...] * norm[:, :1]
    o_ref[...] = jnp.dot(xs, w_ref[...], preferred_element_type=jnp.float32)


def _readout_body(n_valid, n_blocks, rb,
                  agg_ref, deg_ref, ids_ref, b_ref, o_ref, acc_ref):
    i = pl.program_id(0)
    a = agg_ref[0] + agg_ref[1]
    d = deg_ref[0] + deg_ref[1]
    norm = lax.rsqrt(jnp.maximum(d, 1.0))
    lane = lax.broadcasted_iota(jnp.int32, (rb, LANES), 1)
    row = lax.broadcasted_iota(jnp.int32, (rb, LANES), 0) + i * rb
    hn = a * norm + b_ref[...]
    hn = hn + jnp.where(lane == 10, 1.0, 0.0)
    hn = jnp.where(row < n_valid, hn, 0.0)
    gids = lax.broadcasted_iota(jnp.int32, (rb, G), 1)
    oh = jnp.where(ids_ref[...] == gids, 1.0, 0.0)
    contrib = lax.dot_general(
        oh, hn, dimension_numbers=(((0,), (0,)), ((), ())),
        preferred_element_type=jnp.float32,
    )

    @pl.when(i == 0)
    def _():
        acc_ref[...] = contrib

    @pl.when(i > 0)
    def _():
        acc_ref[...] += contrib

    @pl.when(i == n_blocks - 1)
    def _():
        s = acc_ref[...]
        glane = lax.broadcasted_iota(jnp.int32, (G, LANES), 1)
        cnt = jnp.sum(jnp.where(glane == 10, s, 0.0), axis=1, keepdims=True)
        o_ref[...] = s / jnp.maximum(cnt, 1.0)


def kernel(in_feat, edge_index, node_graph_ids, W, b):
    n, d_in = in_feat.shape
    e = edge_index.shape[1]
    c_out = W.shape[1]

    nr = ((n + 1 + 127) // 128) * 128
    e_pad = ((e + NW * CHUNK - 1) // (NW * CHUNK)) * NW * CHUNK
    j_chunks = e_pad // (NW * CHUNK)

    src = jnp.concatenate(
        [edge_index[0], jnp.full((e_pad - e,), n, jnp.int32)]
    ).reshape(NW, j_chunks, CHUNK)
    dst = jnp.concatenate(
        [edge_index[1], jnp.full((e_pad - e,), n, jnp.int32)]
    ).reshape(NW, j_chunks, CHUNK)

    x_pad = jnp.pad(in_feat, ((0, nr - n), (0, 0)))
    w_pad = jnp.pad(W, ((0, 0), (0, LANES - c_out)))
    b_pad = jnp.pad(b, (0, LANES - c_out)).reshape(1, LANES)
    ids2 = jnp.pad(node_graph_ids, (0, nr - n)).reshape(nr, 1)

    deg_out, deg_in = _make_hist_kernel(nr, j_chunks)(src, dst)

    rb = nr // 16
    n_blocks = nr // rb
    h_pad = pl.pallas_call(
        _matmul_body,
        grid=(n_blocks,),
        in_specs=[
            pl.BlockSpec((NC, rb, LANES), lambda i: (0, i, 0)),
            pl.BlockSpec((rb, d_in), lambda i: (i, 0)),
            pl.BlockSpec((d_in, LANES), lambda i: (0, 0)),
        ],
        out_specs=pl.BlockSpec((rb, LANES), lambda i: (i, 0)),
        out_shape=jax.ShapeDtypeStruct((nr, LANES), jnp.float32),
    )(deg_out, x_pad, w_pad)

    agg = _make_scatter_kernel(nr, j_chunks)(h_pad, src, dst)

    out16 = pl.pallas_call(
        functools.partial(_readout_body, n, n_blocks, rb),
        grid=(n_blocks,),
        in_specs=[
            pl.BlockSpec((NC, rb, LANES), lambda i: (0, i, 0)),
            pl.BlockSpec((NC, rb, LANES), lambda i: (0, i, 0)),
            pl.BlockSpec((rb, 1), lambda i: (i, 0)),
            pl.BlockSpec((1, LANES), lambda i: (0, 0)),
        ],
        out_specs=pl.BlockSpec((G, LANES), lambda i: (0, 0)),
        out_shape=jax.ShapeDtypeStruct((G, LANES), jnp.float32),
        scratch_shapes=[pltpu.VMEM((G, LANES), jnp.float32)],
    )(agg, deg_in, ids2, b_pad)

    return out16[:, :c_out]


# speedup vs baseline: 10.4263x; 1.0686x over previous
import functools

import jax
import jax.numpy as jnp
from jax import lax
from jax.experimental import pallas as pl
from jax.experimental.pallas import tpu as pltpu
from jax.experimental.pallas import tpu_sc as plsc

NC = 2
NS = 16
NW = NC * NS
LANES = 16
CHUNK = 128
G = 128


def _sc_mesh():
    return plsc.VectorSubcoreMesh(
        core_axis_name="c", subcore_axis_name="s", num_cores=NC, num_subcores=NS
    )


_SC_PARAMS = pltpu.CompilerParams(use_tc_tiling_on_sc=False)


def _fill(ref, rows, value):
    def body(i, _):
        ref[i] = jnp.full((LANES,), value, jnp.float32)
        return 0

    lax.fori_loop(0, rows, body, 0)


def _make_hist_kernel(nr, j_chunks):
    rt = nr // NS

    @functools.partial(
        pl.kernel,
        out_type=(
            jax.ShapeDtypeStruct((NC, nr, LANES), jnp.float32),
            jax.ShapeDtypeStruct((NC, nr, LANES), jnp.float32),
        ),
        mesh=_sc_mesh(),
        compiler_params=_SC_PARAMS,
        scratch_types=[
            pltpu.VMEM((j_chunks, CHUNK), jnp.int32),
            pltpu.VMEM((j_chunks, CHUNK), jnp.int32),
            pltpu.VMEM((CHUNK, LANES), jnp.float32),
            pltpu.VMEM((rt, LANES), jnp.float32),
            pltpu.VMEM_SHARED((nr, LANES), jnp.float32),
            pltpu.VMEM_SHARED((nr, LANES), jnp.float32),
        ],
    )
    def hist_kernel(src_hbm, dst_hbm, out0_hbm, out1_hbm,
                    src_v, dst_v, ones_v, zero_v, h0_sh, h1_sh):
        c = lax.axis_index("c")
        s = lax.axis_index("s")
        wid = c * NS + s
        base = s * rt

        _fill(ones_v, CHUNK, 1.0)
        _fill(zero_v, rt, 0.0)
        pltpu.sync_copy(zero_v, h0_sh.at[pl.ds(base, rt)])
        pltpu.sync_copy(zero_v, h1_sh.at[pl.ds(base, rt)])
        plsc.subcore_barrier()

        pltpu.sync_copy(src_hbm.at[wid], src_v)
        pltpu.sync_copy(dst_hbm.at[wid], dst_v)

        def chunk(j, _):
            pltpu.sync_copy(ones_v, h0_sh.at[src_v.at[j]], add=True)
            pltpu.sync_copy(ones_v, h1_sh.at[dst_v.at[j]], add=True)
            return 0

        lax.fori_loop(0, j_chunks, chunk, 0)
        plsc.subcore_barrier()

        pltpu.sync_copy(h0_sh.at[pl.ds(base, rt)], out0_hbm.at[c, pl.ds(base, rt)])
        pltpu.sync_copy(h1_sh.at[pl.ds(base, rt)], out1_hbm.at[c, pl.ds(base, rt)])

    return hist_kernel


def _make_scatter_kernel(n_table, nr, j_chunks):
    rt = nr // NS
    jv = j_chunks + 2

    @functools.partial(
        pl.kernel,
        out_type=jax.ShapeDtypeStruct((NC, nr, LANES), jnp.float32),
        mesh=_sc_mesh(),
        compiler_params=_SC_PARAMS,
        scratch_types=[
            pltpu.VMEM((jv, CHUNK), jnp.int32),
            pltpu.VMEM((j_chunks, CHUNK), jnp.int32),
            pltpu.VMEM((CHUNK, LANES), jnp.float32),
            pltpu.VMEM((CHUNK, LANES), jnp.float32),
            pltpu.VMEM((rt, LANES), jnp.float32),
            pltpu.VMEM_SHARED((nr, LANES), jnp.float32),
            pltpu.SemaphoreType.DMA,
            pltpu.SemaphoreType.DMA,
        ],
    )
    def scatter_kernel(h_hbm, src_hbm, dst_hbm, out_hbm,
                       src_v, dst_v, r0_v, r1_v, zero_v, agg_sh, semA, semB):
        c = lax.axis_index("c")
        s = lax.axis_index("s")
        wid = c * NS + s
        base = s * rt

        _fill(zero_v, rt, 0.0)
        izero = jnp.zeros((LANES,), jnp.int32)
        for r in (j_chunks, j_chunks + 1):
            for q in range(CHUNK // LANES):
                src_v[r, pl.ds(q * LANES, LANES)] = izero
        pltpu.sync_copy(zero_v, agg_sh.at[pl.ds(base, rt)])
        plsc.subcore_barrier()

        pltpu.sync_copy(src_hbm.at[wid], src_v.at[pl.ds(0, j_chunks)])
        pltpu.sync_copy(dst_hbm.at[wid], dst_v)

        pltpu.async_copy(h_hbm.at[src_v.at[0]], r0_v, semA)
        pltpu.async_copy(h_hbm.at[src_v.at[1]], r1_v, semB)

        def pair(p, _):
            i = p * 2
            pltpu.make_async_copy(h_hbm.at[src_v.at[i]], r0_v, semA).wait()
            pltpu.sync_copy(r0_v, agg_sh.at[dst_v.at[i]], add=True)
            pltpu.async_copy(h_hbm.at[src_v.at[i + 2]], r0_v, semA)
            pltpu.make_async_copy(h_hbm.at[src_v.at[i + 1]], r1_v, semB).wait()
            pltpu.sync_copy(r1_v, agg_sh.at[dst_v.at[i + 1]], add=True)
            pltpu.async_copy(h_hbm.at[src_v.at[i + 3]], r1_v, semB)
            return 0

        n_pairs = (j_chunks - 1) // 2
        lax.fori_loop(0, n_pairs, pair, 0)

        tail = 2 * n_pairs
        pltpu.make_async_copy(h_hbm.at[src_v.at[tail]], r0_v, semA).wait()
        if tail < j_chunks:
            pltpu.sync_copy(r0_v, agg_sh.at[dst_v.at[tail]], add=True)
        pltpu.make_async_copy(h_hbm.at[src_v.at[tail + 1]], r1_v, semB).wait()

        plsc.subcore_barrier()
        pltpu.sync_copy(agg_sh.at[pl.ds(base, rt)], out_hbm.at[c, pl.ds(base, rt)])

    return scatter_kernel


def _matmul_body(x_ref, w_ref, o_ref):
    o_ref[...] = jnp.dot(x_ref[...], w_ref[...],
                         preferred_element_type=jnp.float32)


def _scale_body(pad_count, rb, deg_ref, h_ref, o_ref):
    i = pl.program_id(0)
    d = deg_ref[0] + deg_ref[1]
    row = lax.broadcasted_iota(jnp.int32, (rb, LANES), 0) + i * rb
    d = jnp.where(row == 0, d - float(pad_count), d)
    norm = lax.rsqrt(jnp.maximum(d, 1.0))
    o_ref[...] = h_ref[...] * norm


def _readout_body(n_valid, n_blocks, rb,
                  agg_ref, deg_ref, ids_ref, b_ref, o_ref, acc_ref):
    i = pl.program_id(0)
    a = agg_ref[0] + agg_ref[1]
    d = deg_ref[0] + deg_ref[1]
    norm = lax.rsqrt(jnp.maximum(d, 1.0))
    lane = lax.broadcasted_iota(jnp.int32, (rb, LANES), 1)
    row = lax.broadcasted_iota(jnp.int32, (rb, LANES), 0) + i * rb
    hn = a * norm + b_ref[...]
    hn = hn + jnp.where(lane == 10, 1.0, 0.0)
    hn = jnp.where(row < n_valid, hn, 0.0)
    gids = lax.broadcasted_iota(jnp.int32, (rb, G), 1)
    oh = jnp.where(ids_ref[...] == gids, 1.0, 0.0)
    contrib = lax.dot_general(
        oh, hn, dimension_numbers=(((0,), (0,)), ((), ())),
        preferred_element_type=jnp.float32,
    )

    @pl.when(i == 0)
    def _():
        acc_ref[...] = contrib

    @pl.when(i > 0)
    def _():
        acc_ref[...] += contrib

    @pl.when(i == n_blocks - 1)
    def _():
        s = acc_ref[...]
        glane = lax.broadcasted_iota(jnp.int32, (G, LANES), 1)
        cnt = jnp.sum(jnp.where(glane == 10, s, 0.0), axis=1, keepdims=True)
        o_ref[...] = s / jnp.maximum(cnt, 1.0)


def kernel(in_feat, edge_index, node_graph_ids, W, b):
    n, d_in = in_feat.shape
    e = edge_index.shape[1]
    c_out = W.shape[1]

    nr = ((n + 1 + 127) // 128) * 128
    e_pad = ((e + NW * CHUNK - 1) // (NW * CHUNK)) * NW * CHUNK
    j_chunks = e_pad // (NW * CHUNK)
    if j_chunks % 2 == 0:
        e_pad += NW * CHUNK
        j_chunks += 1

    src = jnp.concatenate(
        [edge_index[0], jnp.zeros((e_pad - e,), jnp.int32)]
    ).reshape(NW, j_chunks, CHUNK)
    dst = jnp.concatenate(
        [edge_index[1], jnp.full((e_pad - e,), n, jnp.int32)]
    ).reshape(NW, j_chunks, CHUNK)

    w_pad = jnp.pad(W, ((0, 0), (0, LANES - c_out)))
    b_pad = jnp.pad(b, (0, LANES - c_out)).reshape(1, LANES)
    ids2 = jnp.pad(node_graph_ids, (0, nr - n)).reshape(nr, 1)

    deg_out, deg_in = _make_hist_kernel(nr, j_chunks)(src, dst)

    mb = 1000
    h_raw = pl.pallas_call(
        _matmul_body,
        grid=(n // mb,),
        in_specs=[
            pl.BlockSpec((mb, d_in), lambda i: (i, 0)),
            pl.BlockSpec((d_in, LANES), lambda i: (0, 0)),
        ],
        out_specs=pl.BlockSpec((mb, LANES), lambda i: (i, 0)),
        out_shape=jax.ShapeDtypeStruct((n, LANES), jnp.float32),
    )(in_feat, w_pad)

    h_scaled = pl.pallas_call(
        functools.partial(_scale_body, e_pad - e, mb),
        grid=(n // mb,),
        in_specs=[
            pl.BlockSpec((NC, mb, LANES), lambda i: (0, i, 0)),
            pl.BlockSpec((mb, LANES), lambda i: (i, 0)),
        ],
        out_specs=pl.BlockSpec((mb, LANES), lambda i: (i, 0)),
        out_shape=jax.ShapeDtypeStruct((n, LANES), jnp.float32),
    )(deg_out, h_raw)

    agg = _make_scatter_kernel(n, nr, j_chunks)(h_scaled, src, dst)

    rb = nr // 16
    n_blocks = nr // rb
    out16 = pl.pallas_call(
        functools.partial(_readout_body, n, n_blocks, rb),
        grid=(n_blocks,),
        in_specs=[
            pl.BlockSpec((NC, rb, LANES), lambda i: (0, i, 0)),
            pl.BlockSpec((NC, rb, LANES), lambda i: (0, i, 0)),
            pl.BlockSpec((rb, 1), lambda i: (i, 0)),
            pl.BlockSpec((1, LANES), lambda i: (0, 0)),
        ],
        out_specs=pl.BlockSpec((G, LANES), lambda i: (0, 0)),
        out_shape=jax.ShapeDtypeStruct((G, LANES), jnp.float32),
        scratch_shapes=[pltpu.VMEM((G, LANES), jnp.float32)],
    )(agg, deg_in, ids2, b_pad)

    return out16[:, :c_out]
